# baseline (speedup 1.0000x reference)
import dataclasses
import functools

import jax
import jax.numpy as jnp
from jax import lax
from jax.experimental import pallas as pl
from jax.experimental.pallas import tpu as pltpu
from jax.experimental.pallas import tpu_sc as plsc

N = 10000
NPAD = 10240
D = 128
E = 320000
NC, NS = 2, 16
NW = NC * NS
WIN = 64
EPW = 10240
NWIN = EPW // WIN
CW = 16
NCH = NWIN // CW
EPAD = NW * EPW
RPS = NPAD // NS
BLK = 1024


def _vmesh():
    return plsc.VectorSubcoreMesh(core_axis_name="c", subcore_axis_name="s")


def _sc_params():
    cp = pltpu.CompilerParams()
    if "needs_layout_passes" in pltpu.CompilerParams.__dataclass_fields__:
        cp = dataclasses.replace(cp, needs_layout_passes=False)
    return cp


def _deg(dst_flat):
    @functools.partial(
        pl.kernel,
        out_type=jax.ShapeDtypeStruct((NW, NPAD), jnp.float32),
        mesh=_vmesh(),
        compiler_params=_sc_params(),
        scratch_types=[
            pltpu.VMEM((EPW,), jnp.int32),
            pltpu.VMEM((NPAD,), jnp.float32),
        ],
    )
    def deg_kernel(dst_hbm, part_hbm, dstv, hist):
        wid = lax.axis_index("s") * NC + lax.axis_index("c")
        pltpu.sync_copy(dst_hbm.at[wid], dstv)
        zero16 = jnp.zeros((16,), jnp.float32)
        one16 = jnp.full((16,), 1.0, jnp.float32)

        @pl.loop(0, NPAD, step=16)
        def _(i):
            hist[pl.ds(i, 16)] = zero16

        @pl.loop(0, EPW, step=16)
        def _(i):
            idx = dstv[pl.ds(i, 16)]
            plsc.addupdate_scatter(hist, [idx], one16)

        pltpu.sync_copy(hist, part_hbm.at[wid])

    return deg_kernel(dst_flat)


def _agg(hs, srcw, dstw, zeros_nd):
    @functools.partial(
        pl.kernel,
        out_type=jax.ShapeDtypeStruct((NC, NPAD, D), jnp.float32),
        mesh=_vmesh(),
        scratch_types=[
            pltpu.VMEM((CW, WIN), jnp.int32),
            pltpu.VMEM((CW, WIN), jnp.int32),
            pltpu.VMEM((CW, WIN), jnp.int32),
            pltpu.VMEM((CW, WIN), jnp.int32),
            pltpu.VMEM((4 * WIN, D), jnp.float32),
            pltpu.VMEM_SHARED((NPAD, D), jnp.float32),
            pltpu.SemaphoreType.DMA,
            pltpu.SemaphoreType.DMA,
            pltpu.SemaphoreType.DMA,
            pltpu.SemaphoreType.DMA,
            pltpu.SemaphoreType.DMA,
            pltpu.SemaphoreType.DMA,
            pltpu.SemaphoreType.DMA,
            pltpu.SemaphoreType.DMA,
            pltpu.SemaphoreType.DMA,
            pltpu.SemaphoreType.DMA,
        ],
    )
    def agg_kernel(hs_hbm, srcw_hbm, dstw_hbm, zero_hbm, out_hbm,
                   sc0, sc1, dc0, dc1, dbuf, acc,
                   g0, g1, g2, g3, s0, s1, s2, s3, isrc, idst):
        cid = lax.axis_index("c")
        sid = lax.axis_index("s")
        wid = sid * NC + cid
        srcc = (sc0, sc1)
        dstc = (dc0, dc1)
        dbufs = tuple(dbuf.at[pl.ds(b * WIN, WIN)] for b in range(4))
        gsem = (g0, g1, g2, g3)
        ssem = (s0, s1, s2, s3)
        pltpu.sync_copy(zero_hbm.at[pl.ds(sid * RPS, RPS)],
                        acc.at[pl.ds(sid * RPS, RPS)])
        plsc.subcore_barrier()
        pltpu.sync_copy(srcw_hbm.at[wid, pl.ds(0, CW)], sc0)
        pltpu.sync_copy(dstw_hbm.at[wid, pl.ds(0, CW)], dc0)
        pltpu.async_copy(hs_hbm.at[sc0.at[0]], dbufs[0], gsem[0])
        pltpu.async_copy(hs_hbm.at[sc0.at[1]], dbufs[1], gsem[1])

        def slot(sj, b, sc_cur, dc_cur, swait, gnext):
            pltpu.make_async_copy(
                hs_hbm.at[sc_cur.at[sj]], dbufs[b], gsem[b]).wait()
            pltpu.async_copy(dbufs[b], acc.at[dc_cur.at[sj]], ssem[b],
                             add=True)
            b2 = (b + 2) % 4
            if swait is not None:
                sw_ref, sw_row = swait
                pltpu.make_async_copy(
                    dbufs[b2], acc.at[sw_ref.at[sw_row]], ssem[b2]).wait()
            if gnext is not None:
                gn_ref, gn_row = gnext
                pltpu.async_copy(
                    hs_hbm.at[gn_ref.at[gn_row]], dbufs[b2], gsem[b2])

        for c in range(NCH):
            pc = c % 2
            np_ = (c + 1) % 2
            sc_cur, dc_cur = srcc[pc], dstc[pc]
            if c > 0:
                pltpu.make_async_copy(
                    dstw_hbm.at[wid, pl.ds(c * CW, CW)], dc_cur, idst).wait()
            for b in range(2):
                swait = (dstc[np_], CW - 2 + b) if c > 0 else None
                slot(b, b, sc_cur, dc_cur, swait, (sc_cur, 2 + b))
            if c + 1 < NCH:
                pltpu.async_copy(
                    srcw_hbm.at[wid, pl.ds((c + 1) * CW, CW)], srcc[np_], isrc)
                pltpu.async_copy(
                    dstw_hbm.at[wid, pl.ds((c + 1) * CW, CW)], dstc[np_], idst)

            @pl.loop(2, CW - 2, step=4)
            def _(j):
                for bi in range(4):
                    sj = j + bi
                    b = (2 + bi) % 4
                    slot(sj, b, sc_cur, dc_cur,
                         (dc_cur, sj - 2), (sc_cur, sj + 2))

            if c + 1 < NCH:
                pltpu.make_async_copy(
                    srcw_hbm.at[wid, pl.ds((c + 1) * CW, CW)],
                    srcc[np_], isrc).wait()
            for i in range(2):
                sj = CW - 2 + i
                b = sj % 4
                gnext = (srcc[np_], i) if c + 1 < NCH else None
                slot(sj, b, sc_cur, dc_cur, (dc_cur, sj - 2), gnext)

        for i in range(2):
            sj = CW - 2 + i
            b = sj % 4
            pltpu.make_async_copy(
                dbufs[b], acc.at[dstc[(NCH - 1) % 2].at[sj]], ssem[b]).wait()

        plsc.subcore_barrier()
        pltpu.sync_copy(acc.at[pl.ds(sid * RPS, RPS)],
                        out_hbm.at[cid, pl.ds(sid * RPS, RPS)])

    return agg_kernel(hs, srcw, dstw, zeros_nd)


def _h1_body(x_ref, w_ref, h_ref):
    h_ref[...] = jnp.dot(x_ref[...], w_ref[...],
                         preferred_element_type=jnp.float32,
                         precision=lax.Precision.HIGHEST)


def _h1(xp, W1):
    return pl.pallas_call(
        _h1_body,
        grid=(NPAD // BLK,),
        in_specs=[
            pl.BlockSpec((BLK, D), lambda i: (i, 0)),
            pl.BlockSpec((D, D), lambda i: (0, 0)),
        ],
        out_specs=pl.BlockSpec((BLK, D), lambda i: (i, 0)),
        out_shape=jax.ShapeDtypeStruct((NPAD, D), jnp.float32),
    )(xp, W1)


def _scale_body(part_ref, h_ref, hs_ref, dinv_ref):
    i = pl.program_id(0)
    deg = jnp.sum(part_ref[...], axis=0, keepdims=True)
    row = i * BLK + lax.broadcasted_iota(jnp.int32, (1, BLK), 1)
    deg = deg + jnp.where(row < N, 1.0, 0.0)
    dinv = jnp.where(deg > 0, lax.rsqrt(jnp.maximum(deg, 1e-12)), 0.0)
    dinv_c = jnp.transpose(dinv)
    hs_ref[...] = h_ref[...] * dinv_c
    dinv_ref[...] = dinv_c


def _scale(part, h1):
    return pl.pallas_call(
        _scale_body,
        grid=(NPAD // BLK,),
        in_specs=[
            pl.BlockSpec((NW, BLK), lambda i: (0, i)),
            pl.BlockSpec((BLK, D), lambda i: (i, 0)),
        ],
        out_specs=[
            pl.BlockSpec((BLK, D), lambda i: (i, 0)),
            pl.BlockSpec((BLK, 1), lambda i: (i, 0)),
        ],
        out_shape=[
            jax.ShapeDtypeStruct((NPAD, D), jnp.float32),
            jax.ShapeDtypeStruct((NPAD, 1), jnp.float32),
        ],
    )(part, h1)


def _mm2_body(acc_ref, hs_ref, dinv_ref, b_ref, w_ref, out_ref):
    agg = acc_ref[0] + acc_ref[1] + hs_ref[...]
    x2 = jnp.maximum(agg * dinv_ref[...] + b_ref[...], 0.0)
    h = jnp.dot(x2, w_ref[...],
                preferred_element_type=jnp.float32,
                precision=lax.Precision.HIGHEST)
    out_ref[...] = h * dinv_ref[...]


def _mm2(acc, hs, dinv, b, W):
    grid = (NPAD // BLK,)
    return pl.pallas_call(
        _mm2_body,
        grid=grid,
        in_specs=[
            pl.BlockSpec((NC, BLK, D), lambda i: (0, i, 0)),
            pl.BlockSpec((BLK, D), lambda i: (i, 0)),
            pl.BlockSpec((BLK, 1), lambda i: (i, 0)),
            pl.BlockSpec((1, D), lambda i: (0, 0)),
            pl.BlockSpec((D, D), lambda i: (0, 0)),
        ],
        out_specs=pl.BlockSpec((BLK, D), lambda i: (i, 0)),
        out_shape=jax.ShapeDtypeStruct((NPAD, D), jnp.float32),
    )(acc, hs, dinv, b, W)


def _head_body(acc_ref, hs_ref, dinv_ref, b_ref, wh_ref, bh_ref, out_ref):
    agg = acc_ref[0] + acc_ref[1] + hs_ref[...]
    x3 = jnp.maximum(agg * dinv_ref[...] + b_ref[...], 0.0)
    out_ref[...] = jnp.dot(x3, wh_ref[...],
                           preferred_element_type=jnp.float32,
                           precision=lax.Precision.HIGHEST) + bh_ref[...]


def _head(acc, hs, dinv, b, Wh, bh):
    grid = (NPAD // BLK,)
    ncls = Wh.shape[1]
    return pl.pallas_call(
        _head_body,
        grid=grid,
        in_specs=[
            pl.BlockSpec((NC, BLK, D), lambda i: (0, i, 0)),
            pl.BlockSpec((BLK, D), lambda i: (i, 0)),
            pl.BlockSpec((BLK, 1), lambda i: (i, 0)),
            pl.BlockSpec((1, D), lambda i: (0, 0)),
            pl.BlockSpec((D, ncls), lambda i: (0, 0)),
            pl.BlockSpec((1, ncls), lambda i: (0, 0)),
        ],
        out_specs=pl.BlockSpec((BLK, ncls), lambda i: (i, 0)),
        out_shape=jax.ShapeDtypeStruct((NPAD, ncls), jnp.float32),
    )(acc, hs, dinv, b, Wh, bh)


def kernel(x, edge_index, W1, b1, W2, b2, Wh, bh):
    src = edge_index[0].astype(jnp.int32)
    dst = edge_index[1].astype(jnp.int32)
    npad_rows = NPAD - N
    pad = EPAD - E
    padidx = N + (jnp.arange(pad, dtype=jnp.int32) % npad_rows)
    srcw = jnp.concatenate([src, padidx]).reshape(NW, NWIN, WIN)
    dstw = jnp.concatenate([dst, padidx]).reshape(NW, NWIN, WIN)
    dst_flat = dstw.reshape(NW, EPW)

    xp = jnp.concatenate(
        [x, jnp.zeros((npad_rows, D), jnp.float32)], axis=0)
    zeros_nd = jnp.zeros((NPAD, D), jnp.float32)

    part = _deg(dst_flat)
    h1 = _h1(xp, W1)
    hs1, dinv = _scale(part, h1)
    acc1 = _agg(hs1, srcw, dstw, zeros_nd)
    hs2 = _mm2(acc1, hs1, dinv, b1.reshape(1, D), W2)
    acc2 = _agg(hs2, srcw, dstw, zeros_nd)
    out = _head(acc2, hs2, dinv, b2.reshape(1, D), Wh, bh.reshape(1, -1))
    return out[:N]


# speedup vs baseline: 1.0723x; 1.0723x over previous
import dataclasses
import functools

import jax
import jax.numpy as jnp
from jax import lax
from jax.experimental import pallas as pl
from jax.experimental.pallas import tpu as pltpu
from jax.experimental.pallas import tpu_sc as plsc

N = 10000
D = 128
E = 320000
NC, NS = 2, 16
NW = NC * NS
EPW = E // NW
WIN = 125
NWIN = EPW // WIN
CW = 16
NCH = NWIN // CW
RPS = 632
RPS_LAST = N - RPS * (NS - 1)
BLK = 1000
NBLK = N // BLK


def _vmesh():
    return plsc.VectorSubcoreMesh(core_axis_name="c", subcore_axis_name="s")


def _sc_params():
    cp = pltpu.CompilerParams()
    if "needs_layout_passes" in pltpu.CompilerParams.__dataclass_fields__:
        cp = dataclasses.replace(cp, needs_layout_passes=False)
    return cp


def _deg(dst_flat):
    @functools.partial(
        pl.kernel,
        out_type=jax.ShapeDtypeStruct((NW, N), jnp.float32),
        mesh=_vmesh(),
        compiler_params=_sc_params(),
        scratch_types=[
            pltpu.VMEM((EPW,), jnp.int32),
            pltpu.VMEM((N,), jnp.float32),
        ],
    )
    def deg_kernel(dst_hbm, part_hbm, dstv, hist):
        wid = lax.axis_index("s") * NC + lax.axis_index("c")
        pltpu.sync_copy(dst_hbm.at[wid], dstv)
        zero16 = jnp.zeros((16,), jnp.float32)
        one16 = jnp.full((16,), 1.0, jnp.float32)

        @pl.loop(0, N, step=16)
        def _(i):
            hist[pl.ds(i, 16)] = zero16

        @pl.loop(0, EPW, step=16)
        def _(i):
            idx = dstv[pl.ds(i, 16)]
            plsc.addupdate_scatter(hist, [idx], one16)

        pltpu.sync_copy(hist, part_hbm.at[wid])

    return deg_kernel(dst_flat)


def _agg(hs, srcw, dstw, zeros_nd):
    @functools.partial(
        pl.kernel,
        out_type=jax.ShapeDtypeStruct((NC, N, D), jnp.float32),
        mesh=_vmesh(),
        scratch_types=[
            pltpu.VMEM((CW, WIN), jnp.int32),
            pltpu.VMEM((CW, WIN), jnp.int32),
            pltpu.VMEM((CW, WIN), jnp.int32),
            pltpu.VMEM((CW, WIN), jnp.int32),
            pltpu.VMEM((2 * WIN, D), jnp.float32),
            pltpu.VMEM_SHARED((N, D), jnp.float32),
            pltpu.SemaphoreType.DMA,
            pltpu.SemaphoreType.DMA,
            pltpu.SemaphoreType.DMA,
            pltpu.SemaphoreType.DMA,
        ],
    )
    def agg_kernel(hs_hbm, srcw_hbm, dstw_hbm, zero_hbm, out_hbm,
                   sc0, sc1, dc0, dc1, dbuf, acc, g0, g1, isrc, idst):
        cid = lax.axis_index("c")
        sid = lax.axis_index("s")
        wid = sid * NC + cid
        srcc = (sc0, sc1)
        dstc = (dc0, dc1)
        dhalf = (dbuf.at[pl.ds(0, WIN)], dbuf.at[pl.ds(WIN, WIN)])
        gsem = (g0, g1)
        @pl.when(sid < NS - 1)
        def _():
            pltpu.sync_copy(zero_hbm.at[pl.ds(sid * RPS, RPS)],
                            acc.at[pl.ds(sid * RPS, RPS)])

        @pl.when(sid == NS - 1)
        def _():
            pltpu.sync_copy(zero_hbm.at[pl.ds((NS - 1) * RPS, RPS_LAST)],
                            acc.at[pl.ds((NS - 1) * RPS, RPS_LAST)])

        plsc.subcore_barrier()
        pltpu.sync_copy(srcw_hbm.at[wid, pl.ds(0, CW)], sc0)
        pltpu.sync_copy(dstw_hbm.at[wid, pl.ds(0, CW)], dc0)
        pltpu.async_copy(hs_hbm.at[sc0.at[0]], dhalf[0], gsem[0])
        pltpu.async_copy(hs_hbm.at[sc0.at[1]], dhalf[1], gsem[1])

        for c in range(NCH):
            pc = c % 2
            np_ = (c + 1) % 2
            sc_cur, dc_cur = srcc[pc], dstc[pc]
            if c > 0:
                pltpu.make_async_copy(
                    dstw_hbm.at[wid, pl.ds(c * CW, CW)], dc_cur, idst).wait()
            if c + 1 < NCH:
                pltpu.async_copy(
                    srcw_hbm.at[wid, pl.ds((c + 1) * CW, CW)], srcc[np_], isrc)
                pltpu.async_copy(
                    dstw_hbm.at[wid, pl.ds((c + 1) * CW, CW)], dstc[np_], idst)

            @pl.loop(0, CW - 2, step=2)
            def _(j):
                for b in range(2):
                    pltpu.make_async_copy(
                        hs_hbm.at[sc_cur.at[j + b]], dhalf[b], gsem[b]).wait()
                    pltpu.sync_copy(dhalf[b], acc.at[dc_cur.at[j + b]],
                                    add=True)
                    pltpu.async_copy(
                        hs_hbm.at[sc_cur.at[j + b + 2]], dhalf[b], gsem[b])

            if c + 1 < NCH:
                pltpu.make_async_copy(
                    srcw_hbm.at[wid, pl.ds((c + 1) * CW, CW)],
                    srcc[np_], isrc).wait()
            for b in range(2):
                pltpu.make_async_copy(
                    hs_hbm.at[sc_cur.at[CW - 2 + b]], dhalf[b], gsem[b]).wait()
                pltpu.sync_copy(dhalf[b], acc.at[dc_cur.at[CW - 2 + b]],
                                add=True)
                if c + 1 < NCH:
                    pltpu.async_copy(
                        hs_hbm.at[srcc[np_].at[b]], dhalf[b], gsem[b])

        plsc.subcore_barrier()

        @pl.when(sid < NS - 1)
        def _():
            pltpu.sync_copy(acc.at[pl.ds(sid * RPS, RPS)],
                            out_hbm.at[cid, pl.ds(sid * RPS, RPS)])

        @pl.when(sid == NS - 1)
        def _():
            pltpu.sync_copy(acc.at[pl.ds((NS - 1) * RPS, RPS_LAST)],
                            out_hbm.at[cid, pl.ds((NS - 1) * RPS, RPS_LAST)])

    return agg_kernel(hs, srcw, dstw, zeros_nd)


def _h1_body(x_ref, w_ref, h_ref):
    h_ref[...] = jnp.dot(x_ref[...], w_ref[...],
                         preferred_element_type=jnp.float32,
                         precision=lax.Precision.HIGHEST)


def _h1(xp, W1):
    return pl.pallas_call(
        _h1_body,
        grid=(N // BLK,),
        in_specs=[
            pl.BlockSpec((BLK, D), lambda i: (i, 0)),
            pl.BlockSpec((D, D), lambda i: (0, 0)),
        ],
        out_specs=pl.BlockSpec((BLK, D), lambda i: (i, 0)),
        out_shape=jax.ShapeDtypeStruct((N, D), jnp.float32),
    )(xp, W1)


def _scale_body(part_ref, h_ref, hs_ref, dinv_ref):
    deg = jnp.sum(part_ref[0], axis=0, keepdims=True) + 1.0
    dinv = lax.rsqrt(deg)
    dinv_c = jnp.transpose(dinv)
    hs_ref[...] = h_ref[...] * dinv_c
    dinv_ref[...] = dinv_c


def _scale(part, h1):
    return pl.pallas_call(
        _scale_body,
        grid=(N // BLK,),
        in_specs=[
            pl.BlockSpec((1, NW, BLK), lambda i: (i, 0, 0)),
            pl.BlockSpec((BLK, D), lambda i: (i, 0)),
        ],
        out_specs=[
            pl.BlockSpec((BLK, D), lambda i: (i, 0)),
            pl.BlockSpec((BLK, 1), lambda i: (i, 0)),
        ],
        out_shape=[
            jax.ShapeDtypeStruct((N, D), jnp.float32),
            jax.ShapeDtypeStruct((N, 1), jnp.float32),
        ],
    )(part, h1)


def _mm2_body(acc_ref, hs_ref, dinv_ref, b_ref, w_ref, out_ref):
    agg = acc_ref[0] + acc_ref[1] + hs_ref[...]
    x2 = jnp.maximum(agg * dinv_ref[...] + b_ref[...], 0.0)
    h = jnp.dot(x2, w_ref[...],
                preferred_element_type=jnp.float32,
                precision=lax.Precision.HIGHEST)
    out_ref[...] = h * dinv_ref[...]


def _mm2(acc, hs, dinv, b, W):
    return pl.pallas_call(
        _mm2_body,
        grid=(N // BLK,),
        in_specs=[
            pl.BlockSpec((NC, BLK, D), lambda i: (0, i, 0)),
            pl.BlockSpec((BLK, D), lambda i: (i, 0)),
            pl.BlockSpec((BLK, 1), lambda i: (i, 0)),
            pl.BlockSpec((1, D), lambda i: (0, 0)),
            pl.BlockSpec((D, D), lambda i: (0, 0)),
        ],
        out_specs=pl.BlockSpec((BLK, D), lambda i: (i, 0)),
        out_shape=jax.ShapeDtypeStruct((N, D), jnp.float32),
    )(acc, hs, dinv, b, W)


def _head_body(acc_ref, hs_ref, dinv_ref, b_ref, wh_ref, bh_ref, out_ref):
    agg = acc_ref[0] + acc_ref[1] + hs_ref[...]
    x3 = jnp.maximum(agg * dinv_ref[...] + b_ref[...], 0.0)
    out_ref[...] = jnp.dot(x3, wh_ref[...],
                           preferred_element_type=jnp.float32,
                           precision=lax.Precision.HIGHEST) + bh_ref[...]


def _head(acc, hs, dinv, b, Wh, bh):
    ncls = Wh.shape[1]
    return pl.pallas_call(
        _head_body,
        grid=(N // BLK,),
        in_specs=[
            pl.BlockSpec((NC, BLK, D), lambda i: (0, i, 0)),
            pl.BlockSpec((BLK, D), lambda i: (i, 0)),
            pl.BlockSpec((BLK, 1), lambda i: (i, 0)),
            pl.BlockSpec((1, D), lambda i: (0, 0)),
            pl.BlockSpec((D, ncls), lambda i: (0, 0)),
            pl.BlockSpec((1, ncls), lambda i: (0, 0)),
        ],
        out_specs=pl.BlockSpec((BLK, ncls), lambda i: (i, 0)),
        out_shape=jax.ShapeDtypeStruct((N, ncls), jnp.float32),
    )(acc, hs, dinv, b, Wh, bh)


def kernel(x, edge_index, W1, b1, W2, b2, Wh, bh):
    src = edge_index[0].astype(jnp.int32)
    dst = edge_index[1].astype(jnp.int32)
    srcw = src.reshape(NW, NWIN, WIN)
    dstw = dst.reshape(NW, NWIN, WIN)
    dst_flat = dst.reshape(NW, EPW)
    zeros_nd = jnp.zeros((N, D), jnp.float32)

    part = _deg(dst_flat)
    part = jnp.transpose(part.reshape(NW, NBLK, BLK), (1, 0, 2))
    h1 = _h1(x, W1)
    hs1, dinv = _scale(part, h1)
    acc1 = _agg(hs1, srcw, dstw, zeros_nd)
    hs2 = _mm2(acc1, hs1, dinv, b1.reshape(1, D), W2)
    acc2 = _agg(hs2, srcw, dstw, zeros_nd)
    out = _head(acc2, hs2, dinv, b2.reshape(1, D), Wh, bh.reshape(1, -1))
    return out


# speedup vs baseline: 1.1684x; 1.0897x over previous
import dataclasses
import functools

import jax
import jax.numpy as jnp
from jax import lax
from jax.experimental import pallas as pl
from jax.experimental.pallas import tpu as pltpu
from jax.experimental.pallas import tpu_sc as plsc

N = 10000
D = 128
E = 320000
NC, NS = 2, 16
NW = NC * NS
EPW = E // NW
WIN = 125
NWIN = EPW // WIN
CW = 16
NCH = NWIN // CW
RPS = 632
RPS_LAST = N - RPS * (NS - 1)
BLK = 2000
NBLK = N // BLK


def _vmesh():
    return plsc.VectorSubcoreMesh(core_axis_name="c", subcore_axis_name="s")


def _sc_params():
    cp = pltpu.CompilerParams()
    if "needs_layout_passes" in pltpu.CompilerParams.__dataclass_fields__:
        cp = dataclasses.replace(cp, needs_layout_passes=False)
    return cp


def _deg(dst_flat):
    @functools.partial(
        pl.kernel,
        out_type=jax.ShapeDtypeStruct((NW, N), jnp.float32),
        mesh=_vmesh(),
        compiler_params=_sc_params(),
        scratch_types=[
            pltpu.VMEM((EPW,), jnp.int32),
            pltpu.VMEM((N,), jnp.float32),
        ],
    )
    def deg_kernel(dst_hbm, part_hbm, dstv, hist):
        wid = lax.axis_index("s") * NC + lax.axis_index("c")
        pltpu.sync_copy(dst_hbm.at[wid], dstv)
        zero16 = jnp.zeros((16,), jnp.float32)
        one16 = jnp.full((16,), 1.0, jnp.float32)

        @pl.loop(0, N, step=16)
        def _(i):
            hist[pl.ds(i, 16)] = zero16

        @pl.loop(0, EPW, step=16)
        def _(i):
            idx = dstv[pl.ds(i, 16)]
            plsc.addupdate_scatter(hist, [idx], one16)

        pltpu.sync_copy(hist, part_hbm.at[wid])

    return deg_kernel(dst_flat)


def _agg(hs, srcw, dstw, zeros_nd):
    @functools.partial(
        pl.kernel,
        out_type=jax.ShapeDtypeStruct((NC, N, D), jnp.float32),
        mesh=_vmesh(),
        scratch_types=[
            pltpu.VMEM((CW, WIN), jnp.int32),
            pltpu.VMEM((CW, WIN), jnp.int32),
            pltpu.VMEM((CW, WIN), jnp.int32),
            pltpu.VMEM((CW, WIN), jnp.int32),
            pltpu.VMEM((2 * WIN, D), jnp.float32),
            pltpu.VMEM_SHARED((N, D), jnp.float32),
            pltpu.SemaphoreType.DMA,
            pltpu.SemaphoreType.DMA,
            pltpu.SemaphoreType.DMA,
            pltpu.SemaphoreType.DMA,
        ],
    )
    def agg_kernel(hs_hbm, srcw_hbm, dstw_hbm, zero_hbm, out_hbm,
                   sc0, sc1, dc0, dc1, dbuf, acc, g0, g1, isrc, idst):
        cid = lax.axis_index("c")
        sid = lax.axis_index("s")
        wid = sid * NC + cid
        srcc = (sc0, sc1)
        dstc = (dc0, dc1)
        dhalf = (dbuf.at[pl.ds(0, WIN)], dbuf.at[pl.ds(WIN, WIN)])
        gsem = (g0, g1)
        @pl.when(sid < NS - 1)
        def _():
            pltpu.sync_copy(zero_hbm.at[pl.ds(sid * RPS, RPS)],
                            acc.at[pl.ds(sid * RPS, RPS)])

        @pl.when(sid == NS - 1)
        def _():
            pltpu.sync_copy(zero_hbm.at[pl.ds((NS - 1) * RPS, RPS_LAST)],
                            acc.at[pl.ds((NS - 1) * RPS, RPS_LAST)])

        plsc.subcore_barrier()
        pltpu.sync_copy(srcw_hbm.at[wid, pl.ds(0, CW)], sc0)
        pltpu.sync_copy(dstw_hbm.at[wid, pl.ds(0, CW)], dc0)
        pltpu.async_copy(hs_hbm.at[sc0.at[0]], dhalf[0], gsem[0])
        pltpu.async_copy(hs_hbm.at[sc0.at[1]], dhalf[1], gsem[1])

        for c in range(NCH):
            pc = c % 2
            np_ = (c + 1) % 2
            sc_cur, dc_cur = srcc[pc], dstc[pc]
            if c > 0:
                pltpu.make_async_copy(
                    dstw_hbm.at[wid, pl.ds(c * CW, CW)], dc_cur, idst).wait()
            if c + 1 < NCH:
                pltpu.async_copy(
                    srcw_hbm.at[wid, pl.ds((c + 1) * CW, CW)], srcc[np_], isrc)
                pltpu.async_copy(
                    dstw_hbm.at[wid, pl.ds((c + 1) * CW, CW)], dstc[np_], idst)

            @pl.loop(0, CW - 2, step=2)
            def _(j):
                for b in range(2):
                    pltpu.make_async_copy(
                        hs_hbm.at[sc_cur.at[j + b]], dhalf[b], gsem[b]).wait()
                    pltpu.sync_copy(dhalf[b], acc.at[dc_cur.at[j + b]],
                                    add=True)
                    pltpu.async_copy(
                        hs_hbm.at[sc_cur.at[j + b + 2]], dhalf[b], gsem[b])

            if c + 1 < NCH:
                pltpu.make_async_copy(
                    srcw_hbm.at[wid, pl.ds((c + 1) * CW, CW)],
                    srcc[np_], isrc).wait()
            for b in range(2):
                pltpu.make_async_copy(
                    hs_hbm.at[sc_cur.at[CW - 2 + b]], dhalf[b], gsem[b]).wait()
                pltpu.sync_copy(dhalf[b], acc.at[dc_cur.at[CW - 2 + b]],
                                add=True)
                if c + 1 < NCH:
                    pltpu.async_copy(
                        hs_hbm.at[srcc[np_].at[b]], dhalf[b], gsem[b])

        plsc.subcore_barrier()

        @pl.when(sid < NS - 1)
        def _():
            pltpu.sync_copy(acc.at[pl.ds(sid * RPS, RPS)],
                            out_hbm.at[cid, pl.ds(sid * RPS, RPS)])

        @pl.when(sid == NS - 1)
        def _():
            pltpu.sync_copy(acc.at[pl.ds((NS - 1) * RPS, RPS_LAST)],
                            out_hbm.at[cid, pl.ds((NS - 1) * RPS, RPS_LAST)])

    return agg_kernel(hs, srcw, dstw, zeros_nd)


def _h1_body(x_ref, w_ref, h_ref):
    h_ref[...] = jnp.dot(x_ref[...], w_ref[...],
                         preferred_element_type=jnp.float32)


def _h1(xp, W1):
    return pl.pallas_call(
        _h1_body,
        grid=(N // BLK,),
        in_specs=[
            pl.BlockSpec((BLK, D), lambda i: (i, 0)),
            pl.BlockSpec((D, D), lambda i: (0, 0)),
        ],
        out_specs=pl.BlockSpec((BLK, D), lambda i: (i, 0)),
        out_shape=jax.ShapeDtypeStruct((N, D), jnp.float32),
    )(xp, W1)


def _scale_body(part_ref, h_ref, hs_ref, dinv_ref):
    deg = jnp.sum(part_ref[...], axis=0, keepdims=True) + 1.0
    dinv = lax.rsqrt(deg)
    dinv_c = jnp.transpose(dinv)
    hs_ref[...] = h_ref[...] * dinv_c
    dinv_ref[...] = dinv_c


def _scale(part, h1):
    return pl.pallas_call(
        _scale_body,
        in_specs=[
            pl.BlockSpec((NW, N), lambda: (0, 0)),
            pl.BlockSpec((N, D), lambda: (0, 0)),
        ],
        out_specs=[
            pl.BlockSpec((N, D), lambda: (0, 0)),
            pl.BlockSpec((N, 1), lambda: (0, 0)),
        ],
        out_shape=[
            jax.ShapeDtypeStruct((N, D), jnp.float32),
            jax.ShapeDtypeStruct((N, 1), jnp.float32),
        ],
    )(part, h1)


def _mm2_body(acc_ref, hs_ref, dinv_ref, b_ref, w_ref, out_ref):
    agg = acc_ref[0] + acc_ref[1] + hs_ref[...]
    x2 = jnp.maximum(agg * dinv_ref[...] + b_ref[...], 0.0)
    h = jnp.dot(x2, w_ref[...], preferred_element_type=jnp.float32)
    out_ref[...] = h * dinv_ref[...]


def _mm2(acc, hs, dinv, b, W):
    return pl.pallas_call(
        _mm2_body,
        grid=(N // BLK,),
        in_specs=[
            pl.BlockSpec((NC, BLK, D), lambda i: (0, i, 0)),
            pl.BlockSpec((BLK, D), lambda i: (i, 0)),
            pl.BlockSpec((BLK, 1), lambda i: (i, 0)),
            pl.BlockSpec((1, D), lambda i: (0, 0)),
            pl.BlockSpec((D, D), lambda i: (0, 0)),
        ],
        out_specs=pl.BlockSpec((BLK, D), lambda i: (i, 0)),
        out_shape=jax.ShapeDtypeStruct((N, D), jnp.float32),
    )(acc, hs, dinv, b, W)


def _head_body(acc_ref, hs_ref, dinv_ref, b_ref, wh_ref, bh_ref, out_ref):
    agg = acc_ref[0] + acc_ref[1] + hs_ref[...]
    x3 = jnp.maximum(agg * dinv_ref[...] + b_ref[...], 0.0)
    out_ref[...] = jnp.dot(x3, wh_ref[...],
                           preferred_element_type=jnp.float32) + bh_ref[...]


def _head(acc, hs, dinv, b, Wh, bh):
    ncls = Wh.shape[1]
    return pl.pallas_call(
        _head_body,
        grid=(N // BLK,),
        in_specs=[
            pl.BlockSpec((NC, BLK, D), lambda i: (0, i, 0)),
            pl.BlockSpec((BLK, D), lambda i: (i, 0)),
            pl.BlockSpec((BLK, 1), lambda i: (i, 0)),
            pl.BlockSpec((1, D), lambda i: (0, 0)),
            pl.BlockSpec((D, ncls), lambda i: (0, 0)),
            pl.BlockSpec((1, ncls), lambda i: (0, 0)),
        ],
        out_specs=pl.BlockSpec((BLK, ncls), lambda i: (i, 0)),
        out_shape=jax.ShapeDtypeStruct((N, ncls), jnp.float32),
    )(acc, hs, dinv, b, Wh, bh)


def kernel(x, edge_index, W1, b1, W2, b2, Wh, bh):
    src = edge_index[0].astype(jnp.int32)
    dst = edge_index[1].astype(jnp.int32)
    srcw = src.reshape(NW, NWIN, WIN)
    dstw = dst.reshape(NW, NWIN, WIN)
    dst_flat = dst.reshape(NW, EPW)
    zeros_nd = jnp.zeros((N, D), jnp.float32)

    part = _deg(dst_flat)
    h1 = _h1(x, W1)
    hs1, dinv = _scale(part, h1)
    acc1 = _agg(hs1, srcw, dstw, zeros_nd)
    hs2 = _mm2(acc1, hs1, dinv, b1.reshape(1, D), W2)
    acc2 = _agg(hs2, srcw, dstw, zeros_nd)
    out = _head(acc2, hs2, dinv, b2.reshape(1, D), Wh, bh.reshape(1, -1))
    return out


# speedup vs baseline: 1.1879x; 1.0167x over previous
import dataclasses
import functools

import jax
import jax.numpy as jnp
from jax import lax
from jax.experimental import pallas as pl
from jax.experimental.pallas import tpu as pltpu
from jax.experimental.pallas import tpu_sc as plsc

N = 10000
D = 128
E = 320000
NC, NS = 2, 16
NW = NC * NS
EPW = E // NW
WIN = 125
NWIN = EPW // WIN
CW = 16
NCH = NWIN // CW
RPS = 632
RPS_LAST = N - RPS * (NS - 1)
BLK = 2000
NBLK = N // BLK


def _vmesh():
    return plsc.VectorSubcoreMesh(core_axis_name="c", subcore_axis_name="s")


def _sc_params():
    cp = pltpu.CompilerParams()
    if "needs_layout_passes" in pltpu.CompilerParams.__dataclass_fields__:
        cp = dataclasses.replace(cp, needs_layout_passes=False)
    return cp


def _deg(dst_flat):
    @functools.partial(
        pl.kernel,
        out_type=jax.ShapeDtypeStruct((NW, N), jnp.float32),
        mesh=_vmesh(),
        compiler_params=_sc_params(),
        scratch_types=[
            pltpu.VMEM((EPW,), jnp.int32),
            pltpu.VMEM((N,), jnp.float32),
            pltpu.SemaphoreType.DMA,
        ],
    )
    def deg_kernel(dst_hbm, part_hbm, dstv, hist, dsem):
        wid = lax.axis_index("s") * NC + lax.axis_index("c")
        pltpu.async_copy(dst_hbm.at[wid], dstv, dsem)
        zero16 = jnp.zeros((16,), jnp.float32)
        one16 = jnp.full((16,), 1.0, jnp.float32)

        @pl.loop(0, N, step=16)
        def _(i):
            hist[pl.ds(i, 16)] = zero16

        pltpu.make_async_copy(dst_hbm.at[wid], dstv, dsem).wait()

        @pl.loop(0, EPW, step=16)
        def _(i):
            idx = dstv[pl.ds(i, 16)]
            plsc.addupdate_scatter(hist, [idx], one16)

        pltpu.sync_copy(hist, part_hbm.at[wid])

    return deg_kernel(dst_flat)


def _agg(hs, srcw, dstw, zeros_nd):
    @functools.partial(
        pl.kernel,
        out_type=jax.ShapeDtypeStruct((NC, N, D), jnp.float32),
        mesh=_vmesh(),
        scratch_types=[
            pltpu.VMEM((CW, WIN), jnp.int32),
            pltpu.VMEM((CW, WIN), jnp.int32),
            pltpu.VMEM((CW, WIN), jnp.int32),
            pltpu.VMEM((CW, WIN), jnp.int32),
            pltpu.VMEM((2 * WIN, D), jnp.float32),
            pltpu.VMEM_SHARED((N, D), jnp.float32),
            pltpu.SemaphoreType.DMA,
            pltpu.SemaphoreType.DMA,
            pltpu.SemaphoreType.DMA,
            pltpu.SemaphoreType.DMA,
        ],
    )
    def agg_kernel(hs_hbm, srcw_hbm, dstw_hbm, zero_hbm, out_hbm,
                   sc0, sc1, dc0, dc1, dbuf, acc, g0, g1, isrc, idst):
        cid = lax.axis_index("c")
        sid = lax.axis_index("s")
        wid = sid * NC + cid
        srcc = (sc0, sc1)
        dstc = (dc0, dc1)
        dhalf = (dbuf.at[pl.ds(0, WIN)], dbuf.at[pl.ds(WIN, WIN)])
        gsem = (g0, g1)
        pltpu.sync_copy(srcw_hbm.at[wid, pl.ds(0, CW)], sc0)
        pltpu.async_copy(dstw_hbm.at[wid, pl.ds(0, CW)], dc0, idst)
        pltpu.async_copy(hs_hbm.at[sc0.at[0]], dhalf[0], gsem[0])
        pltpu.async_copy(hs_hbm.at[sc0.at[1]], dhalf[1], gsem[1])

        @pl.when(sid < NS - 1)
        def _():
            pltpu.sync_copy(zero_hbm.at[pl.ds(sid * RPS, RPS)],
                            acc.at[pl.ds(sid * RPS, RPS)])

        @pl.when(sid == NS - 1)
        def _():
            pltpu.sync_copy(zero_hbm.at[pl.ds((NS - 1) * RPS, RPS_LAST)],
                            acc.at[pl.ds((NS - 1) * RPS, RPS_LAST)])

        plsc.subcore_barrier()

        for c in range(NCH):
            pc = c % 2
            np_ = (c + 1) % 2
            sc_cur, dc_cur = srcc[pc], dstc[pc]
            pltpu.make_async_copy(
                dstw_hbm.at[wid, pl.ds(c * CW, CW)], dc_cur, idst).wait()
            if c + 1 < NCH:
                pltpu.async_copy(
                    srcw_hbm.at[wid, pl.ds((c + 1) * CW, CW)], srcc[np_], isrc)
                pltpu.async_copy(
                    dstw_hbm.at[wid, pl.ds((c + 1) * CW, CW)], dstc[np_], idst)

            @pl.loop(0, CW - 2, step=2)
            def _(j):
                for b in range(2):
                    pltpu.make_async_copy(
                        hs_hbm.at[sc_cur.at[j + b]], dhalf[b], gsem[b]).wait()
                    pltpu.sync_copy(dhalf[b], acc.at[dc_cur.at[j + b]],
                                    add=True)
                    pltpu.async_copy(
                        hs_hbm.at[sc_cur.at[j + b + 2]], dhalf[b], gsem[b])

            if c + 1 < NCH:
                pltpu.make_async_copy(
                    srcw_hbm.at[wid, pl.ds((c + 1) * CW, CW)],
                    srcc[np_], isrc).wait()
            for b in range(2):
                pltpu.make_async_copy(
                    hs_hbm.at[sc_cur.at[CW - 2 + b]], dhalf[b], gsem[b]).wait()
                pltpu.sync_copy(dhalf[b], acc.at[dc_cur.at[CW - 2 + b]],
                                add=True)
                if c + 1 < NCH:
                    pltpu.async_copy(
                        hs_hbm.at[srcc[np_].at[b]], dhalf[b], gsem[b])

        plsc.subcore_barrier()

        @pl.when(sid < NS - 1)
        def _():
            pltpu.sync_copy(acc.at[pl.ds(sid * RPS, RPS)],
                            out_hbm.at[cid, pl.ds(sid * RPS, RPS)])

        @pl.when(sid == NS - 1)
        def _():
            pltpu.sync_copy(acc.at[pl.ds((NS - 1) * RPS, RPS_LAST)],
                            out_hbm.at[cid, pl.ds((NS - 1) * RPS, RPS_LAST)])

    return agg_kernel(hs, srcw, dstw, zeros_nd)


def _h1_body(x_ref, w_ref, h_ref):
    h_ref[...] = jnp.dot(x_ref[...], w_ref[...],
                         preferred_element_type=jnp.float32)


def _h1(xp, W1):
    return pl.pallas_call(
        _h1_body,
        grid=(N // BLK,),
        in_specs=[
            pl.BlockSpec((BLK, D), lambda i: (i, 0)),
            pl.BlockSpec((D, D), lambda i: (0, 0)),
        ],
        out_specs=pl.BlockSpec((BLK, D), lambda i: (i, 0)),
        out_shape=jax.ShapeDtypeStruct((N, D), jnp.float32),
    )(xp, W1)


def _scale_body(part_ref, h_ref, hs_ref, dinv_ref):
    deg = jnp.sum(part_ref[...], axis=0, keepdims=True) + 1.0
    dinv = lax.rsqrt(deg)
    dinv_c = jnp.transpose(dinv)
    hs_ref[...] = h_ref[...] * dinv_c
    dinv_ref[...] = dinv_c


def _scale(part, h1):
    return pl.pallas_call(
        _scale_body,
        in_specs=[
            pl.BlockSpec((NW, N), lambda: (0, 0)),
            pl.BlockSpec((N, D), lambda: (0, 0)),
        ],
        out_specs=[
            pl.BlockSpec((N, D), lambda: (0, 0)),
            pl.BlockSpec((N, 1), lambda: (0, 0)),
        ],
        out_shape=[
            jax.ShapeDtypeStruct((N, D), jnp.float32),
            jax.ShapeDtypeStruct((N, 1), jnp.float32),
        ],
    )(part, h1)


def _mm2_body(acc_ref, hs_ref, dinv_ref, b_ref, w_ref, out_ref):
    agg = acc_ref[0] + acc_ref[1] + hs_ref[...]
    x2 = jnp.maximum(agg * dinv_ref[...] + b_ref[...], 0.0)
    h = jnp.dot(x2, w_ref[...], preferred_element_type=jnp.float32)
    out_ref[...] = h * dinv_ref[...]


def _mm2(acc, hs, dinv, b, W):
    return pl.pallas_call(
        _mm2_body,
        grid=(N // BLK,),
        in_specs=[
            pl.BlockSpec((NC, BLK, D), lambda i: (0, i, 0)),
            pl.BlockSpec((BLK, D), lambda i: (i, 0)),
            pl.BlockSpec((BLK, 1), lambda i: (i, 0)),
            pl.BlockSpec((1, D), lambda i: (0, 0)),
            pl.BlockSpec((D, D), lambda i: (0, 0)),
        ],
        out_specs=pl.BlockSpec((BLK, D), lambda i: (i, 0)),
        out_shape=jax.ShapeDtypeStruct((N, D), jnp.float32),
    )(acc, hs, dinv, b, W)


def _head_body(acc_ref, hs_ref, dinv_ref, b_ref, wh_ref, bh_ref, out_ref):
    agg = acc_ref[0] + acc_ref[1] + hs_ref[...]
    x3 = jnp.maximum(agg * dinv_ref[...] + b_ref[...], 0.0)
    out_ref[...] = jnp.dot(x3, wh_ref[...],
                           preferred_element_type=jnp.float32) + bh_ref[...]


def _head(acc, hs, dinv, b, Wh, bh):
    ncls = Wh.shape[1]
    return pl.pallas_call(
        _head_body,
        grid=(N // BLK,),
        in_specs=[
            pl.BlockSpec((NC, BLK, D), lambda i: (0, i, 0)),
            pl.BlockSpec((BLK, D), lambda i: (i, 0)),
            pl.BlockSpec((BLK, 1), lambda i: (i, 0)),
            pl.BlockSpec((1, D), lambda i: (0, 0)),
            pl.BlockSpec((D, ncls), lambda i: (0, 0)),
            pl.BlockSpec((1, ncls), lambda i: (0, 0)),
        ],
        out_specs=pl.BlockSpec((BLK, ncls), lambda i: (i, 0)),
        out_shape=jax.ShapeDtypeStruct((N, ncls), jnp.float32),
    )(acc, hs, dinv, b, Wh, bh)


def kernel(x, edge_index, W1, b1, W2, b2, Wh, bh):
    src = edge_index[0].astype(jnp.int32)
    dst = edge_index[1].astype(jnp.int32)
    srcw = src.reshape(NW, NWIN, WIN)
    dstw = dst.reshape(NW, NWIN, WIN)
    dst_flat = dst.reshape(NW, EPW)
    zeros_nd = jnp.zeros((N, D), jnp.float32)

    part = _deg(dst_flat)
    h1 = _h1(x, W1)
    hs1, dinv = _scale(part, h1)
    acc1 = _agg(hs1, srcw, dstw, zeros_nd)
    hs2 = _mm2(acc1, hs1, dinv, b1.reshape(1, D), W2)
    acc2 = _agg(hs2, srcw, dstw, zeros_nd)
    out = _head(acc2, hs2, dinv, b2.reshape(1, D), Wh, bh.reshape(1, -1))
    return out
